# Initial kernel scaffold; baseline (speedup 1.0000x reference)
import jax
import jax.numpy as jnp
from jax.experimental import pallas as pl


def kernel(node_vectors, adj, W1, W2, W3):
    raise NotImplementedError("write your pallas kernel here")


# speedup vs baseline: 1.2345x; 1.2345x over previous
import jax
import jax.numpy as jnp
from jax.experimental import pallas as pl

ROW_BLK = 400


def _xw_kernel(x_ref, w_ref, o_ref):
    o_ref[...] = jnp.dot(x_ref[...], w_ref[...], preferred_element_type=jnp.float32)


def _layer1_kernel(adj_ref, xw1_ref, w23_ref, tw_ref):
    temp = jnp.maximum(
        jnp.dot(adj_ref[...], xw1_ref[...], preferred_element_type=jnp.float32), 0.0)
    tw_ref[...] = jnp.dot(temp, w23_ref[...], preferred_element_type=jnp.float32)


def _layer23_kernel(adj_ref, tw_ref, mv_ref):
    mv_ref[...] = jnp.dot(adj_ref[...], tw_ref[...], preferred_element_type=jnp.float32)


def _decoder_kernel(z_ref, zt_ref, out_ref):
    out_ref[...] = jnp.dot(z_ref[...], zt_ref[...], preferred_element_type=jnp.float32)


def kernel(node_vectors, adj, W1, W2, W3):
    n, _ = node_vectors.shape
    h1 = W1.shape[1]
    h2 = W2.shape[1]
    w23 = jnp.concatenate([W2, W3], axis=1)

    xw1 = pl.pallas_call(
        _xw_kernel,
        out_shape=jax.ShapeDtypeStruct((n, h1), jnp.float32),
    )(node_vectors, W1)

    grid = (n // ROW_BLK,)
    tw = pl.pallas_call(
        _layer1_kernel,
        grid=grid,
        in_specs=[
            pl.BlockSpec((ROW_BLK, n), lambda i: (i, 0)),
            pl.BlockSpec((n, h1), lambda i: (0, 0)),
            pl.BlockSpec((h1, 2 * h2), lambda i: (0, 0)),
        ],
        out_specs=pl.BlockSpec((ROW_BLK, 2 * h2), lambda i: (i, 0)),
        out_shape=jax.ShapeDtypeStruct((n, 2 * h2), jnp.float32),
    )(adj, xw1, w23)

    mv = pl.pallas_call(
        _layer23_kernel,
        grid=grid,
        in_specs=[
            pl.BlockSpec((ROW_BLK, n), lambda i: (i, 0)),
            pl.BlockSpec((n, 2 * h2), lambda i: (0, 0)),
        ],
        out_specs=pl.BlockSpec((ROW_BLK, 2 * h2), lambda i: (i, 0)),
        out_shape=jax.ShapeDtypeStruct((n, 2 * h2), jnp.float32),
    )(adj, tw)

    mean = mv[:, :h2]
    logvar = mv[:, h2:]
    z = mean

    adj_dec = pl.pallas_call(
        _decoder_kernel,
        grid=grid,
        in_specs=[
            pl.BlockSpec((ROW_BLK, h2), lambda i: (i, 0)),
            pl.BlockSpec((h2, n), lambda i: (0, 0)),
        ],
        out_specs=pl.BlockSpec((ROW_BLK, n), lambda i: (i, 0)),
        out_shape=jax.ShapeDtypeStruct((n, n), jnp.float32),
    )(z, z.T)

    return (adj_dec, mean, logvar)


# speedup vs baseline: 1.2558x; 1.0172x over previous
import jax
import jax.numpy as jnp
from jax.experimental import pallas as pl
from jax.experimental.pallas import tpu as pltpu

ROW_BLK = 400


def _layer1_kernel(adj_ref, x_ref, w1_ref, w23_ref, tw_ref, xw1_ref):
    @pl.when(pl.program_id(0) == 0)
    def _():
        xw1_ref[...] = jnp.dot(
            x_ref[...], w1_ref[...], preferred_element_type=jnp.float32)

    temp = jnp.maximum(
        jnp.dot(adj_ref[...], xw1_ref[...], preferred_element_type=jnp.float32), 0.0)
    tw_ref[...] = jnp.dot(temp, w23_ref[...], preferred_element_type=jnp.float32)


def _layer23_kernel(adj_ref, tw_ref, mean_ref, logvar_ref):
    mv = jnp.dot(adj_ref[...], tw_ref[...], preferred_element_type=jnp.float32)
    mean_ref[...] = mv[:, :16]
    logvar_ref[...] = mv[:, 16:]


def _decoder_kernel(z_ref, zt_ref, out_ref):
    out_ref[...] = jnp.dot(z_ref[...], zt_ref[...], preferred_element_type=jnp.float32)


def kernel(node_vectors, adj, W1, W2, W3):
    n, d = node_vectors.shape
    h1 = W1.shape[1]
    h2 = W2.shape[1]
    w23 = jnp.concatenate([W2, W3], axis=1)

    grid = (n // ROW_BLK,)
    tw = pl.pallas_call(
        _layer1_kernel,
        grid=grid,
        in_specs=[
            pl.BlockSpec((ROW_BLK, n), lambda i: (i, 0)),
            pl.BlockSpec((n, d), lambda i: (0, 0)),
            pl.BlockSpec((d, h1), lambda i: (0, 0)),
            pl.BlockSpec((h1, 2 * h2), lambda i: (0, 0)),
        ],
        out_specs=pl.BlockSpec((ROW_BLK, 2 * h2), lambda i: (i, 0)),
        out_shape=jax.ShapeDtypeStruct((n, 2 * h2), jnp.float32),
        scratch_shapes=[pltpu.VMEM((n, h1), jnp.float32)],
    )(adj, node_vectors, W1, w23)

    mean, logvar = pl.pallas_call(
        _layer23_kernel,
        grid=grid,
        in_specs=[
            pl.BlockSpec((ROW_BLK, n), lambda i: (i, 0)),
            pl.BlockSpec((n, 2 * h2), lambda i: (0, 0)),
        ],
        out_specs=[
            pl.BlockSpec((ROW_BLK, h2), lambda i: (i, 0)),
            pl.BlockSpec((ROW_BLK, h2), lambda i: (i, 0)),
        ],
        out_shape=[
            jax.ShapeDtypeStruct((n, h2), jnp.float32),
            jax.ShapeDtypeStruct((n, h2), jnp.float32),
        ],
    )(adj, tw)

    adj_dec = pl.pallas_call(
        _decoder_kernel,
        grid=grid,
        in_specs=[
            pl.BlockSpec((ROW_BLK, h2), lambda i: (i, 0)),
            pl.BlockSpec((h2, n), lambda i: (0, 0)),
        ],
        out_specs=pl.BlockSpec((ROW_BLK, n), lambda i: (i, 0)),
        out_shape=jax.ShapeDtypeStruct((n, n), jnp.float32),
    )(mean, mean.T)

    return (adj_dec, mean, logvar)
